# baseline (speedup 1.0000x reference)
import functools

import jax
import jax.numpy as jnp
from jax import lax
from jax.experimental import pallas as pl
from jax.experimental.pallas import tpu as pltpu
from jax.experimental.pallas import tpu_sc as plsc

_NC = 2
_NS = 16
_NW = _NC * _NS
_CH = 128
_L = 16


@functools.lru_cache(maxsize=None)
def _make_lookup(S, B, VP, D):
    cpb = B // _CH
    nchunks = S * cpb
    assert nchunks % _NW == 0
    cpw = nchunks // _NW
    tpw = cpw * _CH
    epi = 3 + (cpw % 3)
    assert cpw >= epi + 6 and (cpw - 3 - epi) % 3 == 0
    ngrp = _CH // _L

    mesh = plsc.VectorSubcoreMesh(core_axis_name="c", subcore_axis_name="s")

    @functools.partial(
        pl.kernel,
        out_type=jax.ShapeDtypeStruct((S, D, B), jnp.float32),
        mesh=mesh,
        scratch_types=[
            pltpu.VMEM((tpw,), jnp.int32),
            pltpu.VMEM((tpw,), jnp.int32),
            pltpu.VMEM((3, _CH, 2 * D), jnp.float32),
            pltpu.VMEM((3, D, _CH), jnp.float32),
            pltpu.SemaphoreType.DMA((3,)),
            pltpu.SemaphoreType.DMA((3,)),
        ],
        compiler_params=pltpu.CompilerParams(
            use_tc_tiling_on_sc=True,
            needs_layout_passes=False,
            disable_bounds_checks=True,
        ),
    )
    def k(pair_hbm, par_hbm, table_hbm, out_hbm, pairs_v, par_v, rows_v,
          t_v, gsem, ssem):
        wid = lax.axis_index("s") * _NC + lax.axis_index("c")
        cbase = wid * cpw

        pltpu.sync_copy(pair_hbm.at[pl.ds(cbase * _CH, tpw)], pairs_v)
        pltpu.sync_copy(par_hbm.at[pl.ds(cbase * _CH, tpw)], par_v)

        def chunk_sb(kk):
            cid = cbase + kk
            return cid // cpb, (cid % cpb) * _CH

        def start_gather(kk, r):
            pltpu.async_copy(
                table_hbm.at[pairs_v.at[pl.ds(kk * _CH, _CH)]],
                rows_v.at[r],
                gsem.at[r],
            )

        def wait_gather(kk, r):
            pltpu.make_async_copy(
                table_hbm.at[pairs_v.at[pl.ds(kk * _CH, _CH)]],
                rows_v.at[r],
                gsem.at[r],
            ).wait()

        def transpose(kk, r):
            lane = lax.iota(jnp.int32, _L)
            off = kk * _CH

            @plsc.parallel_loop(0, ngrp, unroll=2)
            def _grp(g):
                col = par_v[pl.ds(off + g * _L, _L)]
                jv = lane + g * _L
                cts = [col + _L * t for t in range(D // _L)]
                for d in range(_L):
                    rot = (lane + d) & (_L - 1)
                    for t in range(D // _L):
                        val = plsc.load_gather(
                            rows_v.at[r], [jv, cts[t] + rot]
                        )
                        plsc.store_scatter(
                            t_v.at[r], [rot + _L * t, jv], val
                        )

        def start_store(kk, r):
            s, b0 = chunk_sb(kk)
            pltpu.async_copy(
                t_v.at[r], out_hbm.at[s, :, pl.ds(b0, _CH)], ssem.at[r]
            )

        def wait_store(kk, r):
            s, b0 = chunk_sb(kk)
            pltpu.make_async_copy(
                t_v.at[r], out_hbm.at[s, :, pl.ds(b0, _CH)], ssem.at[r]
            ).wait()

        start_gather(0, 0)
        start_gather(1, 1)
        for kk in range(3):
            r = kk % 3
            wait_gather(kk, r)
            start_gather(kk + 2, (kk + 2) % 3)
            transpose(kk, r)
            start_store(kk, r)

        @pl.loop(3, cpw - epi, step=3)
        def _main(k0):
            for r3 in range(3):
                kk = k0 + r3
                wait_gather(kk, r3)
                start_gather(kk + 2, (kk + 2) % 3)
                wait_store(kk - 3, r3)
                transpose(kk, r3)
                start_store(kk, r3)

        for kk in range(cpw - epi, cpw):
            r = kk % 3
            wait_gather(kk, r)
            if kk + 2 < cpw:
                start_gather(kk + 2, (kk + 2) % 3)
            wait_store(kk - 3, r)
            transpose(kk, r)
            start_store(kk, r)
        for i in range(3):
            kk = cpw - 3 + i
            wait_store(kk, kk % 3)

    return k


def kernel(tokens, wte):
    bsz, seq = tokens.shape
    v, d = wte.shape
    tok = tokens.astype(jnp.int32)
    pair_f = (tok >> 1).T.reshape(-1)
    par_f = ((tok & 1) << 6).T.reshape(-1)
    table2 = wte.reshape(v // 2, 2 * d)
    out_t = _make_lookup(seq, bsz, v // 2, d)(pair_f, par_f, table2)
    return out_t.transpose(2, 0, 1)


# speedup vs baseline: 1.0232x; 1.0232x over previous
import functools

import jax
import jax.numpy as jnp
from jax import lax
from jax.experimental import pallas as pl
from jax.experimental.pallas import tpu as pltpu
from jax.experimental.pallas import tpu_sc as plsc

_NC = 2
_NS = 16
_NW = _NC * _NS
_CH = 128
_L = 16


@functools.lru_cache(maxsize=None)
def _make_lookup(S, B, VP, D):
    cpb = B // _CH
    nchunks = S * cpb
    assert nchunks % _NW == 0
    cpw = nchunks // _NW
    tpw = cpw * _CH
    epi = 3 + (cpw % 3)
    assert cpw >= epi + 6 and (cpw - 3 - epi) % 3 == 0
    ngrp = _CH // _L

    mesh = plsc.VectorSubcoreMesh(core_axis_name="c", subcore_axis_name="s")

    @functools.partial(
        pl.kernel,
        out_type=jax.ShapeDtypeStruct((S, D, B), jnp.float32),
        mesh=mesh,
        scratch_types=[
            pltpu.VMEM((tpw,), jnp.int32),
            pltpu.VMEM((tpw,), jnp.int32),
            pltpu.VMEM((3, _CH, 2 * D), jnp.float32),
            pltpu.VMEM((3, D, _CH), jnp.float32),
            pltpu.SemaphoreType.DMA((3,)),
            pltpu.SemaphoreType.DMA((3,)),
        ],
        compiler_params=pltpu.CompilerParams(
            use_tc_tiling_on_sc=True,
            needs_layout_passes=False,
            disable_bounds_checks=True,
        ),
    )
    def k(pair_hbm, par_hbm, table_hbm, out_hbm, pairs_v, par_v, rows_v,
          t_v, gsem, ssem):
        wid = lax.axis_index("s") * _NC + lax.axis_index("c")
        cbase = wid * cpw

        pltpu.sync_copy(pair_hbm.at[pl.ds(cbase * _CH, tpw)], pairs_v)
        pltpu.sync_copy(par_hbm.at[pl.ds(cbase * _CH, tpw)], par_v)

        def chunk_sb(kk):
            cid = cbase + kk
            return cid // cpb, (cid % cpb) * _CH

        def start_gather(kk, r):
            pltpu.async_copy(
                table_hbm.at[pairs_v.at[pl.ds(kk * _CH, _CH)]],
                rows_v.at[r],
                gsem.at[r],
            )

        def wait_gather(kk, r):
            pltpu.make_async_copy(
                table_hbm.at[pairs_v.at[pl.ds(kk * _CH, _CH)]],
                rows_v.at[r],
                gsem.at[r],
            ).wait()

        def transpose(kk, r):
            lane = lax.iota(jnp.int32, _L)
            off = kk * _CH

            @plsc.parallel_loop(0, ngrp, unroll=2)
            def _grp(g):
                col = par_v[pl.ds(off + g * _L, _L)]
                jv = lane + g * _L
                for t in range(D // _L):
                    ct = col + _L * t
                    for d in range(_L):
                        rot = (lane + d) & (_L - 1)
                        val = plsc.load_gather(
                            rows_v.at[r], [jv, ct + rot]
                        )
                        plsc.store_scatter(
                            t_v.at[r], [rot + _L * t, jv], val
                        )

        def start_store(kk, r):
            s, b0 = chunk_sb(kk)
            pltpu.async_copy(
                t_v.at[r], out_hbm.at[s, :, pl.ds(b0, _CH)], ssem.at[r]
            )

        def wait_store(kk, r):
            s, b0 = chunk_sb(kk)
            pltpu.make_async_copy(
                t_v.at[r], out_hbm.at[s, :, pl.ds(b0, _CH)], ssem.at[r]
            ).wait()

        start_gather(0, 0)
        start_gather(1, 1)
        for kk in range(3):
            r = kk % 3
            wait_gather(kk, r)
            start_gather(kk + 2, (kk + 2) % 3)
            transpose(kk, r)
            start_store(kk, r)

        @pl.loop(3, cpw - epi, step=3)
        def _main(k0):
            for r3 in range(3):
                kk = k0 + r3
                wait_gather(kk, r3)
                start_gather(kk + 2, (kk + 2) % 3)
                wait_store(kk - 3, r3)
                transpose(kk, r3)
                start_store(kk, r3)

        for kk in range(cpw - epi, cpw):
            r = kk % 3
            wait_gather(kk, r)
            if kk + 2 < cpw:
                start_gather(kk + 2, (kk + 2) % 3)
            wait_store(kk - 3, r)
            transpose(kk, r)
            start_store(kk, r)
        for i in range(3):
            kk = cpw - 3 + i
            wait_store(kk, kk % 3)

    return k


def kernel(tokens, wte):
    bsz, seq = tokens.shape
    v, d = wte.shape
    tok = tokens.astype(jnp.int32)
    pair_f = (tok >> 1).T.reshape(-1)
    par_f = ((tok & 1) << 6).T.reshape(-1)
    table2 = wte.reshape(v // 2, 2 * d)
    out_t = _make_lookup(seq, bsz, v // 2, d)(pair_f, par_f, table2)
    return out_t.transpose(2, 0, 1)


# speedup vs baseline: 1.0376x; 1.0140x over previous
import functools

import jax
import jax.numpy as jnp
from jax import lax
from jax.experimental import pallas as pl
from jax.experimental.pallas import tpu as pltpu
from jax.experimental.pallas import tpu_sc as plsc

_NC = 2
_NS = 16
_NW = _NC * _NS
_CH = 128
_L = 16


@functools.lru_cache(maxsize=None)
def _make_lookup(S, B, VP, D):
    cpb = B // _CH
    nchunks = S * cpb
    assert nchunks % _NW == 0
    cpw = nchunks // _NW
    tpw = cpw * _CH
    epi = 3 + (cpw % 3)
    assert cpw >= epi + 6 and (cpw - 3 - epi) % 3 == 0
    ngrp = _CH // _L

    mesh = plsc.VectorSubcoreMesh(core_axis_name="c", subcore_axis_name="s")

    @functools.partial(
        pl.kernel,
        out_type=jax.ShapeDtypeStruct((S, D, B), jnp.float32),
        mesh=mesh,
        scratch_types=[
            pltpu.VMEM((tpw,), jnp.int32),
            pltpu.VMEM((tpw,), jnp.int32),
            pltpu.VMEM((3, _CH, 2 * D), jnp.float32),
            pltpu.VMEM((3, D, _CH), jnp.float32),
            pltpu.SemaphoreType.DMA((3,)),
            pltpu.SemaphoreType.DMA((3,)),
        ],
        compiler_params=pltpu.CompilerParams(
            use_tc_tiling_on_sc=True,
            needs_layout_passes=False,
            disable_bounds_checks=True,
        ),
    )
    def k(pair_hbm, par_hbm, table_hbm, out_hbm, pairs_v, par_v, rows_v,
          t_v, gsem, ssem):
        wid = lax.axis_index("s") * _NC + lax.axis_index("c")
        cbase = wid * cpw

        pltpu.sync_copy(pair_hbm.at[pl.ds(cbase * _CH, tpw)], pairs_v)
        pltpu.sync_copy(par_hbm.at[pl.ds(cbase * _CH, tpw)], par_v)

        def chunk_sb(kk):
            cid = cbase + kk
            return cid // cpb, (cid % cpb) * _CH

        def start_gather(kk, r):
            pltpu.async_copy(
                table_hbm.at[pairs_v.at[pl.ds(kk * _CH, _CH)]],
                rows_v.at[r],
                gsem.at[r],
            )

        def wait_gather(kk, r):
            pltpu.make_async_copy(
                table_hbm.at[pairs_v.at[pl.ds(kk * _CH, _CH)]],
                rows_v.at[r],
                gsem.at[r],
            ).wait()

        def transpose(kk, r):
            lane = lax.iota(jnp.int32, _L)
            off = kk * _CH

            nt = D // _L

            @plsc.parallel_loop(0, ngrp * nt, unroll=2)
            def _grp(gt):
                g = gt // nt
                t = gt % nt
                col = par_v[pl.ds(off + g * _L, _L)]
                jv = lane + g * _L
                ct = col + _L * t
                for d in range(_L):
                    rot = (lane + d) & (_L - 1)
                    val = plsc.load_gather(rows_v.at[r], [jv, ct + rot])
                    plsc.store_scatter(t_v.at[r], [rot + _L * t, jv], val)

        def start_store(kk, r):
            s, b0 = chunk_sb(kk)
            pltpu.async_copy(
                t_v.at[r], out_hbm.at[s, :, pl.ds(b0, _CH)], ssem.at[r]
            )

        def wait_store(kk, r):
            s, b0 = chunk_sb(kk)
            pltpu.make_async_copy(
                t_v.at[r], out_hbm.at[s, :, pl.ds(b0, _CH)], ssem.at[r]
            ).wait()

        start_gather(0, 0)
        start_gather(1, 1)
        for kk in range(3):
            r = kk % 3
            wait_gather(kk, r)
            start_gather(kk + 2, (kk + 2) % 3)
            transpose(kk, r)
            start_store(kk, r)

        @pl.loop(3, cpw - epi, step=3)
        def _main(k0):
            for r3 in range(3):
                kk = k0 + r3
                wait_gather(kk, r3)
                start_gather(kk + 2, (kk + 2) % 3)
                wait_store(kk - 3, r3)
                transpose(kk, r3)
                start_store(kk, r3)

        for kk in range(cpw - epi, cpw):
            r = kk % 3
            wait_gather(kk, r)
            if kk + 2 < cpw:
                start_gather(kk + 2, (kk + 2) % 3)
            wait_store(kk - 3, r)
            transpose(kk, r)
            start_store(kk, r)
        for i in range(3):
            kk = cpw - 3 + i
            wait_store(kk, kk % 3)

    return k


def kernel(tokens, wte):
    bsz, seq = tokens.shape
    v, d = wte.shape
    tok = tokens.astype(jnp.int32)
    pair_f = (tok >> 1).T.reshape(-1)
    par_f = ((tok & 1) << 6).T.reshape(-1)
    table2 = wte.reshape(v // 2, 2 * d)
    out_t = _make_lookup(seq, bsz, v // 2, d)(pair_f, par_f, table2)
    return out_t.transpose(2, 0, 1)
